# Initial kernel scaffold; baseline (speedup 1.0000x reference)
import jax
import jax.numpy as jnp
from jax.experimental import pallas as pl


def kernel(user_id, item_id, user_factors, item_factors, user_bias, item_bias, global_bias):
    raise NotImplementedError("write your pallas kernel here")


# speedup vs baseline: 1.3928x; 1.3928x over previous
import functools

import jax
import jax.numpy as jnp
from jax import lax
from jax.experimental import pallas as pl
from jax.experimental.pallas import tpu as pltpu
from jax.experimental.pallas import tpu_sc as plsc

B = 16384
EMB = 128
NC = 2
NS = 16
NW = NC * NS
PER_W = B // NW
CHUNK = 128
NCHUNK = PER_W // CHUNK
LANES = 16
GROUPS = CHUNK // LANES


def _fm_body(uf, vf, ub, ib, uid, iid, gb,
             pred_out, ctr_out,
             uidc, iidc, urows, vrows, ubias, ibias, predc, ctrc, gbv,
             sem_u, sem_v, sem_ub, sem_ib):
    cid = lax.axis_index("c")
    sid = lax.axis_index("s")
    wid = sid * NC + cid
    base = wid * PER_W

    pltpu.sync_copy(gb, gbv)
    gbvec = gbv[...]
    lane_iota = lax.iota(jnp.int32, LANES)

    def chunk_body(c, carry):
        off = base + c * CHUNK
        pltpu.sync_copy(uid.at[pl.ds(off, CHUNK)], uidc)
        pltpu.sync_copy(iid.at[pl.ds(off, CHUNK)], iidc)
        cu = pltpu.async_copy(uf.at[uidc], urows, sem_u)
        cv = pltpu.async_copy(vf.at[iidc], vrows, sem_v)
        cub = pltpu.async_copy(ub.at[uidc], ubias, sem_ub)
        cib = pltpu.async_copy(ib.at[iidc], ibias, sem_ib)
        cu.wait()
        cv.wait()
        cub.wait()
        cib.wait()

        def group_body(g, inner):
            svec = jnp.zeros((LANES,), jnp.float32)
            for r in range(LANES):
                row = g * LANES + r
                acc = urows[row, pl.ds(0, LANES)] * vrows[row, pl.ds(0, LANES)]
                for j in range(1, EMB // LANES):
                    acc = acc + (urows[row, pl.ds(j * LANES, LANES)]
                                 * vrows[row, pl.ds(j * LANES, LANES)])
                svec = jnp.where(lane_iota == r, jnp.sum(acc), svec)
            p = (svec + ubias[pl.ds(g * LANES, LANES)]
                 + ibias[pl.ds(g * LANES, LANES)] + gbvec)
            predc[pl.ds(g * LANES, LANES)] = p
            ctrc[pl.ds(g * LANES, LANES)] = 1.0 / (1.0 + jnp.exp(-p))
            return inner

        lax.fori_loop(0, GROUPS, group_body, 0)
        pltpu.sync_copy(predc, pred_out.at[pl.ds(off, CHUNK)])
        pltpu.sync_copy(ctrc, ctr_out.at[pl.ds(off, CHUNK)])
        return carry

    lax.fori_loop(0, NCHUNK, chunk_body, 0)


@functools.partial(jax.jit, static_argnames=())
def _fm_call(uid, iid, uf, vf, ub, ib, gb16):
    mesh = plsc.VectorSubcoreMesh(core_axis_name="c", subcore_axis_name="s")
    f32 = jnp.float32
    run = functools.partial(
        pl.kernel,
        mesh=mesh,
        compiler_params=pltpu.CompilerParams(needs_layout_passes=False),
        out_type=[
            jax.ShapeDtypeStruct((B,), f32),
            jax.ShapeDtypeStruct((B,), f32),
        ],
        scratch_types=[
            pltpu.VMEM((CHUNK,), jnp.int32),
            pltpu.VMEM((CHUNK,), jnp.int32),
            pltpu.VMEM((CHUNK, EMB), f32),
            pltpu.VMEM((CHUNK, EMB), f32),
            pltpu.VMEM((CHUNK,), f32),
            pltpu.VMEM((CHUNK,), f32),
            pltpu.VMEM((CHUNK,), f32),
            pltpu.VMEM((CHUNK,), f32),
            pltpu.VMEM((LANES,), f32),
            pltpu.SemaphoreType.DMA,
            pltpu.SemaphoreType.DMA,
            pltpu.SemaphoreType.DMA,
            pltpu.SemaphoreType.DMA,
        ],
    )(_fm_body)
    return run(uf, vf, ub, ib, uid, iid, gb16)


def kernel(user_id, item_id, user_factors, item_factors, user_bias,
           item_bias, global_bias):
    uid = user_id.astype(jnp.int32)
    iid = item_id.astype(jnp.int32)
    gb16 = jnp.broadcast_to(global_bias.astype(jnp.float32), (LANES,))
    pred, ctr = _fm_call(uid, iid, user_factors, item_factors,
                         user_bias, item_bias, gb16)
    return (pred, ctr)


# speedup vs baseline: 1.5551x; 1.1166x over previous
import functools

import jax
import jax.numpy as jnp
from jax import lax
from jax.experimental import pallas as pl
from jax.experimental.pallas import tpu as pltpu
from jax.experimental.pallas import tpu_sc as plsc

B = 16384
EMB = 128
NC = 2
NS = 16
NW = NC * NS
PER_W = B // NW
CHUNK = 128
NCHUNK = PER_W // CHUNK
LANES = 16
GROUPS = CHUNK // LANES


def _fm_body(uf, vf, ub, ib, uid, iid, gb,
             pred_out, ctr_out,
             uidall, iidall, urows, vrows, ubias, ibias, predc, ctrc, gbv,
             sem_u0, sem_u1, sem_v0, sem_v1,
             sem_ub0, sem_ub1, sem_ib0, sem_ib1):
    cid = lax.axis_index("c")
    sid = lax.axis_index("s")
    wid = sid * NC + cid
    base = wid * PER_W

    sem_u = (sem_u0, sem_u1)
    sem_v = (sem_v0, sem_v1)
    sem_ub = (sem_ub0, sem_ub1)
    sem_ib = (sem_ib0, sem_ib1)

    pltpu.sync_copy(gb, gbv)
    pltpu.sync_copy(uid.at[pl.ds(base, PER_W)], uidall)
    pltpu.sync_copy(iid.at[pl.ds(base, PER_W)], iidall)
    gbvec = gbv[...]
    lane_iota = lax.iota(jnp.int32, LANES)

    def issue(c):
        b = c % 2
        idxu = uidall.at[pl.ds(c * CHUNK, CHUNK)]
        idxi = iidall.at[pl.ds(c * CHUNK, CHUNK)]
        return (
            pltpu.async_copy(uf.at[idxu], urows.at[b], sem_u[b]),
            pltpu.async_copy(vf.at[idxi], vrows.at[b], sem_v[b]),
            pltpu.async_copy(ub.at[idxu], ubias.at[b], sem_ub[b]),
            pltpu.async_copy(ib.at[idxi], ibias.at[b], sem_ib[b]),
        )

    descs = [None, None]
    descs[0] = issue(0)

    for c in range(NCHUNK):
        b = c % 2
        if c + 1 < NCHUNK:
            descs[1 - b] = issue(c + 1)
        for d in descs[b]:
            d.wait()

        def group_body(g, inner, b=b):
            svec = jnp.zeros((LANES,), jnp.float32)
            for r in range(LANES):
                row = g * LANES + r
                acc = (urows[b, row, pl.ds(0, LANES)]
                       * vrows[b, row, pl.ds(0, LANES)])
                for j in range(1, EMB // LANES):
                    acc = acc + (urows[b, row, pl.ds(j * LANES, LANES)]
                                 * vrows[b, row, pl.ds(j * LANES, LANES)])
                svec = jnp.where(lane_iota == r, jnp.sum(acc), svec)
            p = (svec + ubias[b, pl.ds(g * LANES, LANES)]
                 + ibias[b, pl.ds(g * LANES, LANES)] + gbvec)
            predc[pl.ds(g * LANES, LANES)] = p
            ctrc[pl.ds(g * LANES, LANES)] = 1.0 / (1.0 + jnp.exp(-p))
            return inner

        lax.fori_loop(0, GROUPS, group_body, 0)
        off = base + c * CHUNK
        pltpu.sync_copy(predc, pred_out.at[pl.ds(off, CHUNK)])
        pltpu.sync_copy(ctrc, ctr_out.at[pl.ds(off, CHUNK)])


@jax.jit
def _fm_call(uid, iid, uf, vf, ub, ib, gb16):
    mesh = plsc.VectorSubcoreMesh(core_axis_name="c", subcore_axis_name="s")
    f32 = jnp.float32
    run = functools.partial(
        pl.kernel,
        mesh=mesh,
        compiler_params=pltpu.CompilerParams(needs_layout_passes=False),
        out_type=[
            jax.ShapeDtypeStruct((B,), f32),
            jax.ShapeDtypeStruct((B,), f32),
        ],
        scratch_types=[
            pltpu.VMEM((PER_W,), jnp.int32),
            pltpu.VMEM((PER_W,), jnp.int32),
            pltpu.VMEM((2, CHUNK, EMB), f32),
            pltpu.VMEM((2, CHUNK, EMB), f32),
            pltpu.VMEM((2, CHUNK), f32),
            pltpu.VMEM((2, CHUNK), f32),
            pltpu.VMEM((CHUNK,), f32),
            pltpu.VMEM((CHUNK,), f32),
            pltpu.VMEM((LANES,), f32),
            pltpu.SemaphoreType.DMA,
            pltpu.SemaphoreType.DMA,
            pltpu.SemaphoreType.DMA,
            pltpu.SemaphoreType.DMA,
            pltpu.SemaphoreType.DMA,
            pltpu.SemaphoreType.DMA,
            pltpu.SemaphoreType.DMA,
            pltpu.SemaphoreType.DMA,
        ],
    )(_fm_body)
    return run(uf, vf, ub, ib, uid, iid, gb16)


def kernel(user_id, item_id, user_factors, item_factors, user_bias,
           item_bias, global_bias):
    uid = user_id.astype(jnp.int32)
    iid = item_id.astype(jnp.int32)
    gb16 = jnp.broadcast_to(global_bias.astype(jnp.float32), (LANES,))
    pred, ctr = _fm_call(uid, iid, user_factors, item_factors,
                         user_bias, item_bias, gb16)
    return (pred, ctr)
